# baseline (speedup 1.0000x reference)
import functools

import jax
import jax.numpy as jnp
from jax import lax
from jax.experimental import pallas as pl
from jax.experimental.pallas import tpu as pltpu
from jax.experimental.pallas import tpu_sc as plsc

_N_NEWTON = 4
_N_SECANT = 2
_L = 16


def _bcast_last(v):
    idx = jnp.full((_L, 1), _L - 1, jnp.int32)
    dnums = lax.GatherDimensionNumbers(
        offset_dims=(), collapsed_slice_dims=(0,), start_index_map=(0,))
    return lax.gather(v, idx, dnums, (1,),
                      mode=lax.GatherScatterMode.PROMISE_IN_BOUNDS)


def _lane_sum(v):
    return _bcast_last(plsc.cumsum(v))


def _lane_max(v):
    return _bcast_last(plsc.cummax(v))


def _make_sc_project(rows, n):
    nslices = n // _L
    info = plsc.get_sparse_core_info()
    nw = info.num_cores * info.num_subcores
    rows_per_w = rows // nw
    mesh = plsc.VectorSubcoreMesh(core_axis_name="c", subcore_axis_name="s")

    @functools.partial(
        pl.kernel,
        mesh=mesh,
        out_type=jax.ShapeDtypeStruct((rows, n), jnp.float32),
        scratch_types=[
            pltpu.VMEM((n,), jnp.float32),
            pltpu.VMEM((n,), jnp.float32),
        ],
        compiler_params=pltpu.CompilerParams(needs_layout_passes=False),
    )
    def sc_project(x_hbm, o_hbm, row_v, out_v):
        wid = lax.axis_index("s") * info.num_cores + lax.axis_index("c")
        base = wid * rows_per_w

        def row_body(r, carry):
            pltpu.sync_copy(x_hbm.at[base + r], row_v)

            def max_body(i, acc):
                return jnp.maximum(acc, row_v[pl.ds(i * _L, _L)])

            acc_m = lax.fori_loop(
                0, nslices, max_body,
                jnp.full((_L,), -jnp.inf, jnp.float32))
            theta = _lane_max(acc_m) - 1.0

            prev_t = theta
            prev_f = jnp.zeros((_L,), jnp.float32)
            for _ in range(_N_NEWTON):
                def newton_body(i, carry2):
                    acc_s, acc_k = carry2
                    xc = row_v[pl.ds(i * _L, _L)]
                    mf = jnp.where(xc > theta, 1.0, 0.0).astype(jnp.float32)
                    return acc_s + xc * mf, acc_k + mf

                acc_s, acc_k = lax.fori_loop(
                    0, nslices, newton_body,
                    (jnp.zeros((_L,), jnp.float32),
                     jnp.zeros((_L,), jnp.float32)))
                s = _lane_sum(acc_s)
                k = _lane_sum(acc_k)
                prev_t = theta
                prev_f = s - k * theta - 1.0
                theta = (s - 1.0) / jnp.maximum(k, 1.0)

            for _ in range(_N_SECANT):
                def sec_body(i, acc):
                    xc = row_v[pl.ds(i * _L, _L)]
                    return acc + jnp.maximum(xc - theta, 0.0)

                acc_f = lax.fori_loop(
                    0, nslices, sec_body, jnp.zeros((_L,), jnp.float32))
                f = _lane_sum(acc_f) - 1.0
                denom = prev_f - f
                step = jnp.where(
                    denom > 0.0,
                    f * (theta - prev_t)
                    / jnp.where(denom == 0.0, 1.0, denom),
                    0.0,
                )
                prev_t = theta
                prev_f = f
                theta = theta + jnp.maximum(step, 0.0)

            def out_body(i, c2):
                xc = row_v[pl.ds(i * _L, _L)]
                out_v[pl.ds(i * _L, _L)] = jnp.maximum(xc - theta, 0.0)
                return c2

            lax.fori_loop(0, nslices, out_body, 0)
            pltpu.sync_copy(out_v, o_hbm.at[base + r])
            return carry

        lax.fori_loop(0, rows_per_w, row_body, 0)

    return sc_project


@jax.jit
def _sc_project_full(x):
    rows, n = x.shape
    return _make_sc_project(rows, n)(x)


def kernel(x):
    return _sc_project_full(x)


# speedup vs baseline: 15.7955x; 15.7955x over previous
import functools

import jax
import jax.numpy as jnp
from jax.experimental import pallas as pl

_N_NEWTON = 4
_N_SECANT = 2


def _simplex_block_kernel(x_ref, o_ref):
    x = x_ref[...]
    theta = jnp.max(x, axis=-1, keepdims=True) - 1.0
    prev_t = theta
    prev_f = jnp.zeros_like(theta)
    for _ in range(_N_NEWTON):
        mf = jnp.where(x > theta, 1.0, 0.0)
        s = jnp.sum(x * mf, axis=-1, keepdims=True)
        k = jnp.sum(mf, axis=-1, keepdims=True)
        prev_t = theta
        prev_f = s - k * theta - 1.0
        theta = (s - 1.0) / jnp.maximum(k, 1.0)
    for _ in range(_N_SECANT):
        f = jnp.sum(jnp.maximum(x - theta, 0.0), axis=-1, keepdims=True) - 1.0
        denom = prev_f - f
        step = jnp.where(
            denom > 0.0,
            f * (theta - prev_t) / jnp.where(denom == 0.0, 1.0, denom),
            0.0,
        )
        prev_t = theta
        prev_f = f
        theta = theta + jnp.maximum(step, 0.0)
    o_ref[...] = jnp.maximum(x - theta, 0.0)


@functools.partial(jax.jit, static_argnames=("block_rows", "interpret"))
def _project(x, block_rows=256, interpret=False):
    rows, n = x.shape
    grid = (rows // block_rows,)
    return pl.pallas_call(
        _simplex_block_kernel,
        grid=grid,
        in_specs=[pl.BlockSpec((block_rows, n), lambda i: (i, 0))],
        out_specs=pl.BlockSpec((block_rows, n), lambda i: (i, 0)),
        out_shape=jax.ShapeDtypeStruct((rows, n), x.dtype),
        interpret=interpret,
    )(x)


def kernel(x):
    return _project(x, block_rows=256)
